# Initial kernel scaffold; baseline (speedup 1.0000x reference)
import jax
import jax.numpy as jnp
from jax.experimental import pallas as pl


def kernel(z, codebook):
    raise NotImplementedError("write your pallas kernel here")


# speedup vs baseline: 4.2119x; 4.2119x over previous
import functools

import jax
import jax.numpy as jnp
from jax import lax
from jax.experimental import pallas as pl
from jax.experimental.pallas import tpu as pltpu

_N_CODES = 512
_CODE_DIM = 64
_H_TILE = 16


def _vq_tc_body(z_ref, cb_ref, idx_ref, zq_ref, acc_ref):
    x = z_ref[0].reshape(_CODE_DIM, _H_TILE * 64)
    cb = cb_ref[...]
    cnorm = jnp.sum(cb * cb, axis=1)
    znorm = jnp.sum(x * x, axis=0)
    s = lax.dot_general(cb, x, (((1,), (0,)), ((), ())),
                        preferred_element_type=jnp.float32)
    dist = (znorm[None, :] - 2.0 * s) + cnorm[:, None]
    m = jnp.min(dist, axis=0)
    kiota = lax.broadcasted_iota(jnp.int32, dist.shape, 0)
    idx = jnp.min(jnp.where(dist == m[None, :], kiota, _N_CODES), axis=0)
    idx_ref[0, 0, :] = idx
    onehot = (kiota == idx[None, :]).astype(jnp.float32)
    zq = lax.dot_general(cb, onehot, (((0,), (0,)), ((), ())),
                         preferred_element_type=jnp.float32)
    zq_ref[0] = (x + (zq - x)).reshape(_CODE_DIM, _H_TILE, 64)

    @pl.when((pl.program_id(0) == 0) & (pl.program_id(1) == 0))
    def _():
        acc_ref[0, 0] = 0.0

    acc_ref[0, 0] += jnp.sum(m)


@jax.jit
def kernel(z, codebook):
    B, C, H, W = z.shape
    nh = H // _H_TILE
    idx3, zq, acc = pl.pallas_call(
        _vq_tc_body,
        grid=(B, nh),
        in_specs=[
            pl.BlockSpec((1, C, _H_TILE, W), lambda b, h: (b, 0, h, 0)),
            pl.BlockSpec((_N_CODES, _CODE_DIM), lambda b, h: (0, 0)),
        ],
        out_specs=[
            pl.BlockSpec((1, 1, _H_TILE * W), lambda b, h, nh=nh: (b * nh + h, 0, 0)),
            pl.BlockSpec((1, C, _H_TILE, W), lambda b, h: (b, 0, h, 0)),
            pl.BlockSpec(memory_space=pltpu.SMEM),
        ],
        out_shape=[
            jax.ShapeDtypeStruct((B * nh, 1, _H_TILE * W), jnp.int32),
            jax.ShapeDtypeStruct((B, C, H, W), jnp.float32),
            jax.ShapeDtypeStruct((1, 1), jnp.float32),
        ],
    )(z, codebook)
    indices = idx3.reshape(B, H * W)
    vq_loss = acc[0, 0] * jnp.float32(1.25 / (B * C * H * W))
    return zq, vq_loss, indices
